# Initial kernel scaffold; baseline (speedup 1.0000x reference)
import jax
import jax.numpy as jnp
from jax.experimental import pallas as pl


def kernel(pha, amp):
    raise NotImplementedError("write your pallas kernel here")


# speedup vs baseline: 113.1505x; 113.1505x over previous
import functools

import numpy as np
import jax
import jax.numpy as jnp
from jax import lax
from jax.experimental import pallas as pl
from jax.experimental.pallas import tpu as pltpu
from jax.experimental.pallas import tpu_sc as plsc

_NB = 18
_NCOL = 9
_F = 8
_T = 1024
_NW = 32
_ROWS = _F * _NCOL * _NB
_CHUNKS = _T // 16

_CUTS = [float(v) for v in np.linspace(-np.pi, np.pi, _NB + 1).astype(np.float32)[1:_NB]]


def _sc_histogram(phat, ampt):
    mesh = plsc.VectorSubcoreMesh(core_axis_name="c", subcore_axis_name="s")

    @functools.partial(
        pl.kernel,
        out_type=jax.ShapeDtypeStruct((_NW, _ROWS * 16), jnp.float32),
        mesh=mesh,
        compiler_params=pltpu.CompilerParams(needs_layout_passes=False),
        scratch_types=[
            pltpu.VMEM((_F, _T), jnp.float32),
            pltpu.VMEM((_F, _T), jnp.float32),
            pltpu.VMEM((_ROWS * 16,), jnp.float32),
        ],
    )
    def k(pha_hbm, amp_hbm, out_hbm, pha_v, amp_v, hist_v):
        wid = lax.axis_index("s") * 2 + lax.axis_index("c")
        pltpu.sync_copy(pha_hbm.at[wid], pha_v)
        pltpu.sync_copy(amp_hbm.at[wid], amp_v)

        zero16 = jnp.zeros((16,), jnp.float32)
        for r in range(_ROWS):
            hist_v[pl.ds(r * 16, 16)] = zero16

        ones16 = jnp.ones((16,), jnp.float32)
        lane = lax.iota(jnp.int32, 16)

        def chunk_body(t, carry):
            off = t * 16
            amps = [amp_v[fa, pl.ds(off, 16)] for fa in range(_F)]
            for fp in range(_F):
                x = pha_v[fp, pl.ds(off, 16)]
                b = jnp.zeros((16,), jnp.int32)
                for c in _CUTS:
                    b = b + (x > c).astype(jnp.int32)
                base = b * 16 + (lane + fp * (_NCOL * _NB * 16))
                for fa in range(_F):
                    plsc.addupdate_scatter(hist_v, [base + fa * (_NB * 16)], amps[fa])
                plsc.addupdate_scatter(hist_v, [base + _F * (_NB * 16)], ones16)
            return carry

        lax.fori_loop(0, _CHUNKS, chunk_body, 0)
        pltpu.sync_copy(hist_v, out_hbm.at[wid])

    return k(phat, ampt)


def _mi_body(h_ref, o_ref):
    eps = jnp.float32(1e-9)
    h = h_ref[...]
    sums = jnp.sum(h, axis=-1)
    amp_sums = sums[:, :, 0:_F, :]
    counts = sums[:, :, _F:_F + 1, :]
    means = amp_sums / (counts + eps)
    probs = means / (jnp.sum(means, axis=-1, keepdims=True) + eps)
    ent = jnp.sum(probs * jnp.log(probs + eps), axis=-1)
    nb = jnp.float32(_NB)
    mi = (jnp.log(nb + eps) + ent) / jnp.log(nb)
    o_ref[...] = jnp.mean(mi, axis=0)[None]


def kernel(pha, amp):
    B, C, F, S, T = pha.shape
    phat = jnp.transpose(pha[0], (0, 2, 1, 3)).reshape(C * S, F, T)
    ampt = jnp.transpose(amp[0], (0, 2, 1, 3)).reshape(C * S, F, T)

    hist = _sc_histogram(phat, ampt)
    hist5 = hist.reshape(C * S, F, _NCOL, _NB, 16)

    mi = pl.pallas_call(
        _mi_body,
        grid=(C,),
        in_specs=[pl.BlockSpec((S, F, _NCOL, _NB, 16), lambda i: (i, 0, 0, 0, 0))],
        out_specs=pl.BlockSpec((1, F, F), lambda i: (i, 0, 0)),
        out_shape=jax.ShapeDtypeStruct((C, F, F), jnp.float32),
    )(hist5)
    return mi.reshape(B, C, F, F)


# speedup vs baseline: 115.0864x; 1.0171x over previous
import functools

import numpy as np
import jax
import jax.numpy as jnp
from jax import lax
from jax.experimental import pallas as pl
from jax.experimental.pallas import tpu as pltpu
from jax.experimental.pallas import tpu_sc as plsc

_NB = 18
_NCOL = 9
_F = 8
_T = 1024
_NW = 32
_ROWS = _F * _NCOL * _NB
_CHUNKS = _T // 16

_CUTS = [float(v) for v in np.linspace(-np.pi, np.pi, _NB + 1).astype(np.float32)[1:_NB]]


def _sc_histogram(phat, ampt):
    mesh = plsc.VectorSubcoreMesh(core_axis_name="c", subcore_axis_name="s")

    @functools.partial(
        pl.kernel,
        out_type=jax.ShapeDtypeStruct((_NW, _ROWS * 16), jnp.float32),
        mesh=mesh,
        compiler_params=pltpu.CompilerParams(needs_layout_passes=False),
        scratch_types=[
            pltpu.VMEM((_F, _T), jnp.float32),
            pltpu.VMEM((_F, _T), jnp.float32),
            pltpu.VMEM((_ROWS * 16,), jnp.float32),
        ],
    )
    def k(pha_hbm, amp_hbm, out_hbm, pha_v, amp_v, hist_v):
        wid = lax.axis_index("s") * 2 + lax.axis_index("c")
        ci = wid // 4
        si = wid - ci * 4
        pltpu.sync_copy(pha_hbm.at[0, ci, :, si, :], pha_v)
        pltpu.sync_copy(amp_hbm.at[0, ci, :, si, :], amp_v)

        zero16 = jnp.zeros((16,), jnp.float32)
        for r in range(_ROWS):
            hist_v[pl.ds(r * 16, 16)] = zero16

        ones16 = jnp.ones((16,), jnp.float32)
        lane = lax.iota(jnp.int32, 16)

        def chunk_body(t, carry):
            off = t * 16
            amps = [amp_v[fa, pl.ds(off, 16)] for fa in range(_F)]
            for fp in range(_F):
                x = pha_v[fp, pl.ds(off, 16)]
                b = jnp.zeros((16,), jnp.int32)
                for c in _CUTS:
                    b = b + (x > c).astype(jnp.int32)
                base = b * 16 + (lane + fp * (_NCOL * _NB * 16))
                for fa in range(_F):
                    plsc.addupdate_scatter(hist_v, [base + fa * (_NB * 16)], amps[fa])
                plsc.addupdate_scatter(hist_v, [base + _F * (_NB * 16)], ones16)
            return carry

        lax.fori_loop(0, _CHUNKS, chunk_body, 0)
        pltpu.sync_copy(hist_v, out_hbm.at[wid])

    return k(phat, ampt)


def _mi_body(h_ref, o_ref):
    eps = jnp.float32(1e-9)
    h = h_ref[...]
    sums = jnp.sum(h, axis=-1)
    amp_sums = sums[:, :, 0:_F, :]
    counts = sums[:, :, _F:_F + 1, :]
    means = amp_sums / (counts + eps)
    probs = means / (jnp.sum(means, axis=-1, keepdims=True) + eps)
    ent = jnp.sum(probs * jnp.log(probs + eps), axis=-1)
    nb = jnp.float32(_NB)
    mi = (jnp.log(nb + eps) + ent) / jnp.log(nb)
    o_ref[...] = jnp.mean(mi, axis=0)[None]


def kernel(pha, amp):
    B, C, F, S, T = pha.shape
    hist = _sc_histogram(pha, amp)
    hist5 = hist.reshape(C * S, F, _NCOL, _NB, 16)

    mi = pl.pallas_call(
        _mi_body,
        grid=(C,),
        in_specs=[pl.BlockSpec((S, F, _NCOL, _NB, 16), lambda i: (i, 0, 0, 0, 0))],
        out_specs=pl.BlockSpec((1, F, F), lambda i: (i, 0, 0)),
        out_shape=jax.ShapeDtypeStruct((C, F, F), jnp.float32),
    )(hist5)
    return mi.reshape(B, C, F, F)


# speedup vs baseline: 192.2413x; 1.6704x over previous
import functools

import numpy as np
import jax
import jax.numpy as jnp
from jax import lax
from jax.experimental import pallas as pl
from jax.experimental.pallas import tpu as pltpu
from jax.experimental.pallas import tpu_sc as plsc

_NB = 18
_NCOL = 9
_F = 8
_T = 1024
_NW = 32
_ROWS = _F * _NCOL * _NB
_CHUNKS = _T // 16

_CUTS = [float(v) for v in np.linspace(-np.pi, np.pi, _NB + 1).astype(np.float32)[1:_NB]]


def _sc_histogram(phat, ampt):
    mesh = plsc.VectorSubcoreMesh(core_axis_name="c", subcore_axis_name="s")

    @functools.partial(
        pl.kernel,
        out_type=jax.ShapeDtypeStruct((_NW, _ROWS * 16), jnp.float32),
        mesh=mesh,
        compiler_params=pltpu.CompilerParams(needs_layout_passes=False),
        scratch_types=[
            pltpu.VMEM((_F, _T), jnp.float32),
            pltpu.VMEM((_F, _T), jnp.float32),
            pltpu.VMEM((_ROWS * 16,), jnp.float32),
        ],
    )
    def k(pha_hbm, amp_hbm, out_hbm, pha_v, amp_v, hist_v):
        wid = lax.axis_index("s") * 2 + lax.axis_index("c")
        ci = wid // 4
        si = wid - ci * 4
        pltpu.sync_copy(pha_hbm.at[0, ci, :, si, :], pha_v)
        pltpu.sync_copy(amp_hbm.at[0, ci, :, si, :], amp_v)

        zero16 = jnp.zeros((16,), jnp.float32)
        for r in range(_ROWS):
            hist_v[pl.ds(r * 16, 16)] = zero16

        ones16 = jnp.ones((16,), jnp.float32)
        lane = lax.iota(jnp.int32, 16)

        def chunk_body(t, carry):
            off = t * 16
            amps = [amp_v[fa, pl.ds(off, 16)] for fa in range(_F)]
            for fp in range(_F):
                x = pha_v[fp, pl.ds(off, 16)]
                b = jnp.zeros((16,), jnp.int32)
                for c in _CUTS:
                    b = b + (x > c).astype(jnp.int32)
                base = b * 128 + (lane + fp * 16)
                for fa in range(_F):
                    plsc.addupdate_scatter(hist_v, [base + fa * (_NB * 128)], amps[fa])
                plsc.addupdate_scatter(hist_v, [base + _F * (_NB * 128)], ones16)
            return carry

        lax.fori_loop(0, _CHUNKS, chunk_body, 0)
        pltpu.sync_copy(hist_v, out_hbm.at[wid])

    return k(phat, ampt)


def _mi_body(h_ref, o_ref):
    eps = jnp.float32(1e-9)
    h = h_ref[...]
    hm = h.reshape(_NW * (_NCOL * _NB), 128)
    li = jax.lax.broadcasted_iota(jnp.int32, (128, _F), 0)
    gi = jax.lax.broadcasted_iota(jnp.int32, (128, _F), 1)
    G = (li // 16 == gi).astype(jnp.float32)
    s8 = jax.lax.dot_general(hm, G, (((1,), (0,)), ((), ())),
                             precision=jax.lax.Precision.HIGHEST)
    s8 = s8.reshape(_NW, _NCOL * _NB, _F)
    counts = s8[:, _F * _NB:, :]
    wi = jax.lax.broadcasted_iota(jnp.int32, (_F, _NW), 1)
    ci = jax.lax.broadcasted_iota(jnp.int32, (_F, _NW), 0)
    A = jnp.where(wi // 4 == ci, jnp.float32(0.25), jnp.float32(0.0))
    nb = jnp.float32(_NB)
    for fa in range(_F):
        sums = s8[:, fa * _NB:(fa + 1) * _NB, :]
        means = sums / (counts + eps)
        probs = means / (jnp.sum(means, axis=1, keepdims=True) + eps)
        ent = jnp.sum(probs * jnp.log(probs + eps), axis=1)
        mi = (jnp.log(nb + eps) + ent) / jnp.log(nb)
        o_ref[fa] = jax.lax.dot_general(
            A, mi, (((1,), (0,)), ((), ())),
            precision=jax.lax.Precision.HIGHEST)


def kernel(pha, amp):
    B, C, F, S, T = pha.shape
    hist = _sc_histogram(pha, amp)
    h3 = hist.reshape(_NW, _NCOL * _NB, 128)

    mi = pl.pallas_call(
        _mi_body,
        out_shape=jax.ShapeDtypeStruct((F, C, F), jnp.float32),
    )(h3)
    return jnp.transpose(mi, (1, 2, 0)).reshape(B, C, F, F)
